# Initial kernel scaffold; baseline (speedup 1.0000x reference)
import jax
import jax.numpy as jnp
from jax.experimental import pallas as pl


def kernel(x, edge_index, W1, b1, W2, b2):
    raise NotImplementedError("write your pallas kernel here")


# speedup vs baseline: 12.4594x; 12.4594x over previous
import functools

import jax
import jax.numpy as jnp
from jax import lax
from jax.experimental import pallas as pl
from jax.experimental.pallas import tpu as pltpu
from jax.experimental.pallas import tpu_sc as plsc

N = 10000
E = 320000
D = 128

NC = 2
NS = 16
NW = NC * NS

C = 128
CH = 79
EPAD = NW * CH * C
RT = 632
R = NS * RT

_mesh = plsc.VectorSubcoreMesh(core_axis_name="c", subcore_axis_name="s")


@functools.partial(
    pl.kernel,
    out_type=jax.ShapeDtypeStruct((NC * R,), jnp.float32),
    mesh=_mesh,
    scratch_types=[
        pltpu.VMEM((CH, C), jnp.int32),
        pltpu.VMEM((C,), jnp.float32),
        pltpu.VMEM((RT,), jnp.float32),
        pltpu.VMEM_SHARED((R,), jnp.float32),
    ],
)
def _sc_degree(cols_hbm, ones_hbm, zeros_hbm, out_hbm, colv, onesv, stg, acc):
    c = lax.axis_index("c")
    s = lax.axis_index("s")
    w = c * NS + s
    tbase = s * RT
    pltpu.sync_copy(cols_hbm.at[w], colv)
    pltpu.sync_copy(ones_hbm, onesv)
    pltpu.sync_copy(zeros_hbm, stg)
    pltpu.sync_copy(stg, acc.at[pl.ds(tbase, RT)])
    plsc.subcore_barrier()
    for j in range(CH):
        pltpu.sync_copy(onesv, acc.at[colv.at[j]], add=True)
    plsc.subcore_barrier()
    pltpu.sync_copy(acc.at[pl.ds(tbase, RT)], stg)
    pltpu.sync_copy(stg, out_hbm.at[pl.ds(c * R + tbase, RT)])


@functools.partial(
    pl.kernel,
    out_type=jax.ShapeDtypeStruct((NC, R, D), jnp.float32),
    mesh=_mesh,
    scratch_types=[
        pltpu.VMEM((CH, C), jnp.int32),
        pltpu.VMEM((CH, C), jnp.int32),
        pltpu.VMEM((C, D), jnp.float32),
        pltpu.VMEM_SHARED((R, D), jnp.float32),
        pltpu.SemaphoreType.DMA,
    ],
)
def _sc_aggregate(rows_hbm, cols_hbm, hs_hbm, zeros_hbm, out_hbm,
                  rowv, colv, rbuf, acc, sem):
    c = lax.axis_index("c")
    s = lax.axis_index("s")
    w = c * NS + s
    tbase = s * RT
    pltpu.sync_copy(rows_hbm.at[w], rowv)
    pltpu.sync_copy(cols_hbm.at[w], colv)
    pltpu.sync_copy(zeros_hbm, rbuf)
    for k in range(RT // C):
        pltpu.sync_copy(rbuf, acc.at[pl.ds(tbase + k * C, C)])
    rem = RT % C
    if rem:
        pltpu.sync_copy(rbuf.at[pl.ds(0, rem)],
                        acc.at[pl.ds(tbase + (RT // C) * C, rem)])
    plsc.subcore_barrier()
    for j in range(CH):
        pltpu.async_copy(hs_hbm.at[rowv.at[j]], rbuf, sem).wait()
        pltpu.sync_copy(rbuf, acc.at[colv.at[j]], add=True)
    plsc.subcore_barrier()
    pltpu.sync_copy(acc.at[pl.ds(tbase, RT)], out_hbm.at[c, pl.ds(tbase, RT)])


BR = 1000


def _tc_scale_matmul_body(deg_ref, x_ref, w_ref, o_ref):
    dis = lax.rsqrt(deg_ref[...] + 1.0)
    h = jnp.dot(x_ref[...], w_ref[...], preferred_element_type=jnp.float32)
    o_ref[...] = h * dis


def _tc_mid_body(deg_ref, sp_ref, hs_ref, b_ref, w_ref, o_ref):
    dis = lax.rsqrt(deg_ref[...] + 1.0)
    tot = sp_ref[0] + sp_ref[1] + hs_ref[...]
    h1 = jnp.maximum(tot * dis + b_ref[...], 0.0)
    o_ref[...] = jnp.dot(h1, w_ref[...],
                         preferred_element_type=jnp.float32) * dis


def _tc_final_body(deg_ref, sp_ref, hs_ref, b_ref, o_ref):
    dis = lax.rsqrt(deg_ref[...] + 1.0)
    o_ref[...] = (sp_ref[0] + sp_ref[1] + hs_ref[...]) * dis + b_ref[...]


_deg_spec = pl.BlockSpec((BR, D), lambda i: (i, 0))
_sp_spec = pl.BlockSpec((NC, BR, D), lambda i: (0, i, 0))
_row_spec = pl.BlockSpec((BR, D), lambda i: (i, 0))
_w_spec = pl.BlockSpec((D, D), lambda i: (0, 0))
_b_spec = pl.BlockSpec((1, D), lambda i: (0, 0))
_out_shape = jax.ShapeDtypeStruct((N, D), jnp.float32)

_tc_scale_matmul = pl.pallas_call(
    _tc_scale_matmul_body,
    grid=(N // BR,),
    in_specs=[_deg_spec, _row_spec, _w_spec],
    out_specs=_row_spec,
    out_shape=_out_shape,
)

_tc_mid = pl.pallas_call(
    _tc_mid_body,
    grid=(N // BR,),
    in_specs=[_deg_spec, _sp_spec, _row_spec, _b_spec, _w_spec],
    out_specs=_row_spec,
    out_shape=_out_shape,
)

_tc_final = pl.pallas_call(
    _tc_final_body,
    grid=(N // BR,),
    in_specs=[_deg_spec, _sp_spec, _row_spec, _b_spec],
    out_specs=_row_spec,
    out_shape=_out_shape,
)


def kernel(x, edge_index, W1, b1, W2, b2):
    row = edge_index[0].astype(jnp.int32)
    col = edge_index[1].astype(jnp.int32)
    pad = EPAD - E
    rows3 = jnp.concatenate(
        [row, jnp.zeros((pad,), jnp.int32)]).reshape(NW, CH, C)
    cols3 = jnp.concatenate(
        [col, jnp.full((pad,), N, jnp.int32)]).reshape(NW, CH, C)

    ones1 = jnp.ones((C,), jnp.float32)
    zeros_rt = jnp.zeros((RT,), jnp.float32)
    zerosD = jnp.zeros((C, D), jnp.float32)
    b1r = b1.reshape(1, D)
    b2r = b2.reshape(1, D)

    dgp = _sc_degree(cols3, ones1, zeros_rt)
    deg_b = jnp.broadcast_to((dgp[:R] + dgp[R:])[:, None], (R, D))

    hs1 = _tc_scale_matmul(deg_b, x, W1)
    s1 = _sc_aggregate(rows3, cols3, hs1, zerosD)
    hs2 = _tc_mid(deg_b, s1, hs1, b1r, W2)
    s2 = _sc_aggregate(rows3, cols3, hs2, zerosD)
    return _tc_final(deg_b, s2, hs2, b2r)
